# Initial kernel scaffold; baseline (speedup 1.0000x reference)
import jax
import jax.numpy as jnp
from jax.experimental import pallas as pl


def kernel(x_curr, cond, num_x_weight, num_x_bias, num_cond_weight, num_cond_bias, x_cat_emb, cond_cat_emb):
    raise NotImplementedError("write your pallas kernel here")


# speedup vs baseline: 1.0546x; 1.0546x over previous
import functools

import jax
import jax.numpy as jnp
from jax import lax
from jax.experimental import pallas as pl
from jax.experimental.pallas import tpu as pltpu
from jax.experimental.pallas import tpu_sc as plsc

B = 16384
D = 64
XN = 13
CN = 4
XF, XS = 26, 100
CF, CS = 8, 50
NXT = XN + XF
NCT = CN + CF
NROWS = XF * XS + CF * CS

BSZ = 256
GRID = B // BSZ


def _tc_body(x_ref, c_ref, wx_ref, bx_ref, wc_ref, bc_ref,
             tx_ref, tc_ref, gx_ref, gc_ref):
    x = x_ref[...]
    c = c_ref[...]
    for f in range(XN):
        tx_ref[:, f, :] = (
            x[:, f:f + 1] * wx_ref[:, f * D:(f + 1) * D]
            + bx_ref[:, f * D:(f + 1) * D])
    for f in range(CN):
        tc_ref[:, f * D:(f + 1) * D] = (
            c[:, f:f + 1] * wc_ref[:, f * D:(f + 1) * D]
            + bc_ref[:, f * D:(f + 1) * D])
    iota_x = lax.broadcasted_iota(jnp.int32, (BSZ, XS), 1)
    cols = []
    for f in range(XF):
        ch = x[:, XN + XS * f: XN + XS * (f + 1)]
        m = jnp.max(ch, axis=1, keepdims=True)
        cand = jnp.where(ch == m, iota_x + XS * f, NROWS)
        cols.append(jnp.min(cand, axis=1, keepdims=True))
    gx_ref[...] = jnp.concatenate(cols, axis=1)
    iota_c = lax.broadcasted_iota(jnp.int32, (BSZ, CS), 1)
    cols = []
    for f in range(CF):
        ch = c[:, CN + CS * f: CN + CS * (f + 1)]
        m = jnp.max(ch, axis=1, keepdims=True)
        cand = jnp.where(ch == m, iota_c + (XF * XS + CS * f), NROWS)
        cols.append(jnp.min(cand, axis=1, keepdims=True))
    gc_ref[...] = jnp.concatenate(cols, axis=1)


_tc_encode = pl.pallas_call(
    _tc_body,
    grid=(GRID,),
    in_specs=[
        pl.BlockSpec((BSZ, XN + XF * XS), lambda i: (i, 0)),
        pl.BlockSpec((BSZ, CN + CF * CS), lambda i: (i, 0)),
        pl.BlockSpec((1, XN * D), lambda i: (0, 0)),
        pl.BlockSpec((1, XN * D), lambda i: (0, 0)),
        pl.BlockSpec((1, CN * D), lambda i: (0, 0)),
        pl.BlockSpec((1, CN * D), lambda i: (0, 0)),
    ],
    out_specs=[
        pl.BlockSpec((BSZ, 16, D), lambda i: (i, 0, 0)),
        pl.BlockSpec((BSZ, CN * D), lambda i: (i, 0)),
        pl.BlockSpec((BSZ, XF), lambda i: (i, 0)),
        pl.BlockSpec((BSZ, CF), lambda i: (i, 0)),
    ],
    out_shape=[
        jax.ShapeDtypeStruct((B, NXT, D), jnp.float32),
        jax.ShapeDtypeStruct((B, NCT * D), jnp.float32),
        jax.ShapeDtypeStruct((B, XF), jnp.int32),
        jax.ShapeDtypeStruct((B, CF), jnp.int32),
    ],
)


NWC, NWS = 2, 16
NW = NWC * NWS
BPW = B // NW
CB = 16
NCH = BPW // CB


def _sc_body(gx_hbm, gc_hbm, emb_hbm, ox_hbm, oc_hbm,
             idxx, idxc, bufx, bufc, semx, semc, semo):
    wid = lax.axis_index("s") * NWC + lax.axis_index("c")
    base = wid * BPW

    @pl.loop(0, NCH)
    def _chunk(ch):
        b0 = base + ch * CB
        pltpu.sync_copy(gx_hbm.at[pl.ds(b0 * XF, CB * XF)], idxx)
        pltpu.sync_copy(gc_hbm.at[pl.ds(b0 * CF, CB * CF)], idxc)
        cpx = pltpu.async_copy(emb_hbm.at[idxx], bufx, semx)
        cpc = pltpu.async_copy(emb_hbm.at[idxc], bufc, semc)
        cpx.wait()
        cpc.wait()
        for j in range(CB):
            pltpu.async_copy(
                bufx.at[pl.ds(j * XF, XF), :],
                ox_hbm.at[pl.ds((b0 + j) * NXT + XN, XF), :], semo)
            pltpu.async_copy(
                bufc.at[pl.ds(j * CF, CF), :],
                oc_hbm.at[pl.ds((b0 + j) * NCT + CN, CF), :], semo)
        pltpu.make_async_copy(
            ox_hbm.at[pl.ds(0, CB * (XF + CF)), :],
            ox_hbm.at[pl.ds(0, CB * (XF + CF)), :], semo).wait()


@functools.lru_cache(maxsize=None)
def _make_sc_fill():
    return pl.kernel(
        _sc_body,
        out_type=(),
        mesh=plsc.VectorSubcoreMesh(core_axis_name="c", subcore_axis_name="s"),
        compiler_params=pltpu.CompilerParams(use_tc_tiling_on_sc=False),
        scratch_types=[
            pltpu.VMEM((CB * XF,), jnp.int32),
            pltpu.VMEM((CB * CF,), jnp.int32),
            pltpu.VMEM((CB * XF, D), jnp.float32),
            pltpu.VMEM((CB * CF, D), jnp.float32),
            pltpu.SemaphoreType.DMA,
            pltpu.SemaphoreType.DMA,
            pltpu.SemaphoreType.DMA,
        ],
    )


def kernel(x_curr, cond, num_x_weight, num_x_bias, num_cond_weight,
           num_cond_bias, x_cat_emb, cond_cat_emb):
    emb_all = jnp.concatenate(
        [x_cat_emb.reshape(XF * XS, D), cond_cat_emb.reshape(CF * CS, D)],
        axis=0)
    tokx0, tokc0, gx, gc = _tc_encode(
        x_curr, cond,
        num_x_weight.reshape(1, XN * D), num_x_bias.reshape(1, XN * D),
        num_cond_weight.reshape(1, CN * D), num_cond_bias.reshape(1, CN * D))
    rx = jax.new_ref(tokx0.reshape(B * NXT, D))
    rc = jax.new_ref(tokc0.reshape(B * NCT, D))
    _make_sc_fill()(gx.reshape(B * XF), gc.reshape(B * CF), emb_all, rx, rc)
    return rx[...].reshape(B, NXT, D), rc[...].reshape(B, NCT, D)
